# baseline (speedup 1.0000x reference)
import functools
import jax
import jax.numpy as jnp
from jax import lax
from jax.experimental import pallas as pl
from jax.experimental.pallas import tpu as pltpu
from jax.experimental.pallas import tpu_sc as plsc

_B, _N, _D, _K = 16, 4096, 256, 4
_NC, _NS, _L = 2, 16, 16
_NW = _NC * _NS
_NPW = _N // _NW
_NCH = 8
_CHUNKS = _NPW // _NCH
_GD = _D // _L
_ZS = 3
_AS = 2

_mesh = plsc.VectorSubcoreMesh(core_axis_name="c", subcore_axis_name="s")


@functools.partial(
    pl.kernel,
    out_type=jax.ShapeDtypeStruct((_B, _N, 2 * _D), jnp.float32),
    mesh=_mesh,
    scratch_types=[
        pltpu.VMEM((_ZS, _B, _NCH, _D), jnp.float32),
        pltpu.VMEM((_AS, _K, _NCH, _D), jnp.float32),
        pltpu.VMEM((_L,), jnp.int32),
        pltpu.SemaphoreType.DMA((_ZS,)),
        pltpu.SemaphoreType.DMA((_ZS,)),
        pltpu.SemaphoreType.DMA((_AS,)),
    ],
)
def _mixer(z_hbm, ch_hbm, out_hbm, z_ring, acc_ring, ch_v, in_sems, cp_sems,
           ag_sems):
    wid = lax.axis_index("s") * _NC + lax.axis_index("c")
    n0 = wid * _NPW

    pltpu.sync_copy(ch_hbm, ch_v)
    ch = ch_v[...]
    ks = [ch[b] for b in range(_B)]
    wks = []
    for k in range(_K):
        cnt = jnp.int32(0)
        for b in range(_B):
            cnt = cnt + jnp.where(ks[b] == k, 1, 0)
        wk = jnp.float32(1.0)
        for c in range(2, _B + 1):
            wk = jnp.where(cnt == c, jnp.float32(1.0 / c), wk)
        wks.append(wk)
    ws = []
    for b in range(_B):
        wb = wks[0]
        for k in range(1, _K):
            wb = jnp.where(ks[b] == k, wks[k], wb)
        ws.append(wb)

    zero = jnp.zeros((_L,), jnp.float32)

    def start_in(nb, zs):
        pltpu.async_copy(z_hbm.at[:, pl.ds(nb, _NCH), :], z_ring.at[zs],
                         in_sems.at[zs])

    def wait_in(zs):
        pltpu.make_async_copy(z_hbm.at[:, pl.ds(0, _NCH), :], z_ring.at[zs],
                              in_sems.at[zs]).wait()

    def start_copy_out(nb, zs):
        pltpu.async_copy(z_ring.at[zs],
                         out_hbm.at[:, pl.ds(nb, _NCH), pl.ds(0, _D)],
                         cp_sems.at[zs])

    def wait_copy_out(zs):
        pltpu.make_async_copy(z_ring.at[zs],
                              out_hbm.at[:, pl.ds(0, _NCH), pl.ds(0, _D)],
                              cp_sems.at[zs]).wait()

    def start_aggr_out(nb, asl):
        for b in range(_B):
            pltpu.async_copy(acc_ring.at[asl, ks[b]],
                             out_hbm.at[b, pl.ds(nb, _NCH), pl.ds(_D, _D)],
                             ag_sems.at[asl])

    def wait_aggr_out(asl):
        pltpu.make_async_copy(out_hbm.at[:, pl.ds(0, _NCH), pl.ds(_D, _D)],
                              z_ring.at[0], ag_sems.at[asl]).wait()

    def chunk_body(c, _):
        zs = lax.rem(c, _ZS)
        asl = lax.rem(c, _AS)
        nb = n0 + c * _NCH
        wait_in(zs)

        nzs = lax.rem(c + 1, _ZS)

        @pl.when(c >= _ZS - 1)
        def _():
            wait_copy_out(nzs)

        @pl.when(c < _CHUNKS - 1)
        def _():
            start_in(nb + _NCH, nzs)

        @pl.when(c >= _AS)
        def _():
            wait_aggr_out(asl)

        @plsc.parallel_loop(0, _K * _NCH * _D, step=_L, unroll=16)
        def _(p):
            k = lax.shift_right_logical(p, 11)
            rem = lax.bitwise_and(p, _NCH * _D - 1)
            r = lax.shift_right_logical(rem, 8)
            col = pl.multiple_of(lax.bitwise_and(rem, _D - 1), _L)
            acc_ring[asl, k, r, pl.ds(col, _L)] = zero

        for b in range(_B):
            wb = ws[b]
            kb = ks[b]

            @plsc.parallel_loop(0, _NCH * _D, step=_L, unroll=16)
            def _(p, b=b, wb=wb, kb=kb):
                r = lax.shift_right_logical(p, 8)
                col = pl.multiple_of(lax.bitwise_and(p, _D - 1), _L)
                seg = z_ring[zs, b, r, pl.ds(col, _L)]
                plsc.addupdate(acc_ring.at[asl, kb, r, pl.ds(col, _L)],
                               seg * wb)

        start_copy_out(nb, zs)
        start_aggr_out(nb, asl)
        return 0

    start_in(n0, 0)
    lax.fori_loop(0, _CHUNKS, chunk_body, 0)

    for c in range(_CHUNKS - (_ZS - 1), _CHUNKS):
        wait_copy_out(c % _ZS)
    for asl in range(_AS):
        wait_aggr_out(asl)


def kernel(z, ch_ids):
    zs = z.reshape(_B, _N, _D)
    return _mixer(zs, ch_ids)


# speedup vs baseline: 1.0198x; 1.0198x over previous
import functools
import jax
import jax.numpy as jnp
from jax import lax
from jax.experimental import pallas as pl
from jax.experimental.pallas import tpu as pltpu
from jax.experimental.pallas import tpu_sc as plsc

_B, _N, _D, _K = 16, 4096, 256, 4
_NC, _NS, _L = 2, 16, 16
_NW = _NC * _NS
_NPW = _N // _NW
_NCH = 8
_CHUNKS = _NPW // _NCH
_GD = _D // _L
_ZS = 3
_AS = 2

_mesh = plsc.VectorSubcoreMesh(core_axis_name="c", subcore_axis_name="s")


@functools.partial(
    pl.kernel,
    out_type=jax.ShapeDtypeStruct((_B, _N, 2 * _D), jnp.float32),
    mesh=_mesh,
    scratch_types=[
        pltpu.VMEM((_ZS, _B, _NCH, _D), jnp.float32),
        pltpu.VMEM((_AS, _K, _NCH, _D), jnp.float32),
        pltpu.VMEM((_L,), jnp.int32),
        pltpu.SemaphoreType.DMA((_ZS,)),
        pltpu.SemaphoreType.DMA((_ZS,)),
        pltpu.SemaphoreType.DMA((_AS,)),
    ],
)
def _mixer(z_hbm, ch_hbm, out_hbm, z_ring, acc_ring, ch_v, in_sems, cp_sems,
           ag_sems):
    wid = lax.axis_index("s") * _NC + lax.axis_index("c")
    n0 = wid * _NPW

    pltpu.sync_copy(ch_hbm, ch_v)
    ch = ch_v[...]
    ks = [ch[b] for b in range(_B)]
    wks = []
    for k in range(_K):
        cnt = jnp.int32(0)
        for b in range(_B):
            cnt = cnt + jnp.where(ks[b] == k, 1, 0)
        wk = jnp.float32(1.0)
        for c in range(2, _B + 1):
            wk = jnp.where(cnt == c, jnp.float32(1.0 / c), wk)
        wks.append(wk)
    ws = []
    for b in range(_B):
        wb = wks[0]
        for k in range(1, _K):
            wb = jnp.where(ks[b] == k, wks[k], wb)
        ws.append(wb)

    zero = jnp.zeros((_L,), jnp.float32)

    def start_in(nb, zs):
        pltpu.async_copy(z_hbm.at[:, pl.ds(nb, _NCH), :], z_ring.at[zs],
                         in_sems.at[zs])

    def wait_in(zs):
        pltpu.make_async_copy(z_hbm.at[:, pl.ds(0, _NCH), :], z_ring.at[zs],
                              in_sems.at[zs]).wait()

    def start_copy_out(nb, zs):
        pltpu.async_copy(z_ring.at[zs],
                         out_hbm.at[:, pl.ds(nb, _NCH), pl.ds(0, _D)],
                         cp_sems.at[zs])

    def wait_copy_out(zs):
        pltpu.make_async_copy(z_ring.at[zs],
                              out_hbm.at[:, pl.ds(0, _NCH), pl.ds(0, _D)],
                              cp_sems.at[zs]).wait()

    def start_aggr_out(nb, asl):
        for b in range(_B):
            pltpu.async_copy(acc_ring.at[asl, ks[b]],
                             out_hbm.at[b, pl.ds(nb, _NCH), pl.ds(_D, _D)],
                             ag_sems.at[asl])

    def wait_aggr_out(asl):
        pltpu.make_async_copy(out_hbm.at[:, pl.ds(0, _NCH), pl.ds(_D, _D)],
                              z_ring.at[0], ag_sems.at[asl]).wait()

    def chunk_body(c, _):
        zs = lax.rem(c, _ZS)
        asl = lax.rem(c, _AS)
        nb = n0 + c * _NCH
        wait_in(zs)

        nzs = lax.rem(c + 1, _ZS)

        @pl.when(c >= _ZS - 1)
        def _():
            wait_copy_out(nzs)

        @pl.when(jnp.logical_and(c >= 1, c < _CHUNKS - 1))
        def _():
            start_in(nb + _NCH, nzs)

        start_copy_out(nb, zs)

        @pl.when(c >= _AS)
        def _():
            wait_aggr_out(asl)

        @plsc.parallel_loop(0, _K * _NCH * _D, step=_L, unroll=8)
        def _(p):
            k = lax.shift_right_logical(p, 11)
            rem = lax.bitwise_and(p, _NCH * _D - 1)
            r = lax.shift_right_logical(rem, 8)
            col = pl.multiple_of(lax.bitwise_and(rem, _D - 1), _L)
            acc_ring[asl, k, r, pl.ds(col, _L)] = zero

        for b in range(_B):
            wb = ws[b]
            kb = ks[b]

            @plsc.parallel_loop(0, _NCH * _D, step=_L, unroll=8)
            def _(p, b=b, wb=wb, kb=kb):
                r = lax.shift_right_logical(p, 8)
                col = pl.multiple_of(lax.bitwise_and(p, _D - 1), _L)
                seg = z_ring[zs, b, r, pl.ds(col, _L)]
                plsc.addupdate(acc_ring.at[asl, kb, r, pl.ds(col, _L)],
                               seg * wb)

        start_aggr_out(nb, asl)
        return 0

    start_in(n0, 0)
    start_in(n0 + _NCH, 1)
    lax.fori_loop(0, _CHUNKS, chunk_body, 0)

    for c in range(_CHUNKS - (_ZS - 1), _CHUNKS):
        wait_copy_out(c % _ZS)
    for asl in range(_AS):
        wait_aggr_out(asl)


def kernel(z, ch_ids):
    zs = z.reshape(_B, _N, _D)
    return _mixer(zs, ch_ids)


# speedup vs baseline: 1.0238x; 1.0038x over previous
import functools
import jax
import jax.numpy as jnp
from jax import lax
from jax.experimental import pallas as pl
from jax.experimental.pallas import tpu as pltpu
from jax.experimental.pallas import tpu_sc as plsc

_B, _N, _D, _K = 16, 4096, 256, 4
_NC, _NS, _L = 2, 16, 16
_NW = _NC * _NS
_NPW = _N // _NW
_NCH = 8
_CHUNKS = _NPW // _NCH
_GD = _D // _L
_ZS = 3
_AS = 2

_mesh = plsc.VectorSubcoreMesh(core_axis_name="c", subcore_axis_name="s")


@functools.partial(
    pl.kernel,
    out_type=jax.ShapeDtypeStruct((_B, _N, 2 * _D), jnp.float32),
    mesh=_mesh,
    scratch_types=[
        pltpu.VMEM((_ZS, _B, _NCH, _D), jnp.float32),
        pltpu.VMEM((_AS, _K, _NCH, _D), jnp.float32),
        pltpu.VMEM((_L,), jnp.int32),
        pltpu.SemaphoreType.DMA((_ZS,)),
        pltpu.SemaphoreType.DMA((_ZS,)),
        pltpu.SemaphoreType.DMA((_AS,)),
    ],
)
def _mixer(z_hbm, ch_hbm, out_hbm, z_ring, acc_ring, ch_v, in_sems, cp_sems,
           ag_sems):
    wid = lax.axis_index("s") * _NC + lax.axis_index("c")
    n0 = wid * _NPW

    pltpu.async_copy(z_hbm.at[:, pl.ds(n0, _NCH), :], z_ring.at[0],
                     in_sems.at[0])
    pltpu.async_copy(z_hbm.at[:, pl.ds(n0 + _NCH, _NCH), :], z_ring.at[1],
                     in_sems.at[1])

    pltpu.sync_copy(ch_hbm, ch_v)
    ch = ch_v[...]
    ks = [ch[b] for b in range(_B)]
    wks = []
    for k in range(_K):
        cnt = jnp.int32(0)
        for b in range(_B):
            cnt = cnt + jnp.where(ks[b] == k, 1, 0)
        wk = jnp.float32(1.0)
        for c in range(2, _B + 1):
            wk = jnp.where(cnt == c, jnp.float32(1.0 / c), wk)
        wks.append(wk)
    ws = []
    for b in range(_B):
        wb = wks[0]
        for k in range(1, _K):
            wb = jnp.where(ks[b] == k, wks[k], wb)
        ws.append(wb)

    zero = jnp.zeros((_L,), jnp.float32)

    def start_in(nb, zs):
        pltpu.async_copy(z_hbm.at[:, pl.ds(nb, _NCH), :], z_ring.at[zs],
                         in_sems.at[zs])

    def wait_in(zs):
        pltpu.make_async_copy(z_hbm.at[:, pl.ds(0, _NCH), :], z_ring.at[zs],
                              in_sems.at[zs]).wait()

    def start_copy_out(nb, zs):
        pltpu.async_copy(z_ring.at[zs],
                         out_hbm.at[:, pl.ds(nb, _NCH), pl.ds(0, _D)],
                         cp_sems.at[zs])

    def wait_copy_out(zs):
        pltpu.make_async_copy(z_ring.at[zs],
                              out_hbm.at[:, pl.ds(0, _NCH), pl.ds(0, _D)],
                              cp_sems.at[zs]).wait()

    def start_aggr_out(nb, asl):
        for b in range(_B):
            pltpu.async_copy(acc_ring.at[asl, ks[b]],
                             out_hbm.at[b, pl.ds(nb, _NCH), pl.ds(_D, _D)],
                             ag_sems.at[asl])

    def wait_aggr_out(asl):
        pltpu.make_async_copy(out_hbm.at[:, pl.ds(0, _NCH), pl.ds(_D, _D)],
                              z_ring.at[0], ag_sems.at[asl]).wait()

    def chunk_body(c, _):
        zs = lax.rem(c, _ZS)
        asl = lax.rem(c, _AS)
        nb = n0 + c * _NCH
        wait_in(zs)

        nzs = lax.rem(c + 1, _ZS)

        @pl.when(c >= _ZS - 1)
        def _():
            wait_copy_out(nzs)

        @pl.when(jnp.logical_and(c >= 1, c < _CHUNKS - 1))
        def _():
            start_in(nb + _NCH, nzs)

        start_copy_out(nb, zs)

        @pl.when(c >= _AS)
        def _():
            wait_aggr_out(asl)

        @plsc.parallel_loop(0, _K * _NCH * _D, step=_L, unroll=8)
        def _(p):
            k = lax.shift_right_logical(p, 11)
            rem = lax.bitwise_and(p, _NCH * _D - 1)
            r = lax.shift_right_logical(rem, 8)
            col = pl.multiple_of(lax.bitwise_and(rem, _D - 1), _L)
            acc_ring[asl, k, r, pl.ds(col, _L)] = zero

        for b in range(_B):
            wb = ws[b]
            kb = ks[b]

            @plsc.parallel_loop(0, _NCH * _D, step=_L, unroll=8)
            def _(p, b=b, wb=wb, kb=kb):
                r = lax.shift_right_logical(p, 8)
                col = pl.multiple_of(lax.bitwise_and(p, _D - 1), _L)
                seg = z_ring[zs, b, r, pl.ds(col, _L)]
                plsc.addupdate(acc_ring.at[asl, kb, r, pl.ds(col, _L)],
                               seg * wb)

        start_aggr_out(nb, asl)
        return 0

    lax.fori_loop(0, _CHUNKS, chunk_body, 0)

    for c in range(_CHUNKS - (_ZS - 1), _CHUNKS):
        wait_copy_out(c % _ZS)
    for asl in range(_AS):
        wait_aggr_out(asl)


def kernel(z, ch_ids):
    zs = z.reshape(_B, _N, _D)
    return _mixer(zs, ch_ids)
